# Initial kernel scaffold; baseline (speedup 1.0000x reference)
import jax
import jax.numpy as jnp
from jax.experimental import pallas as pl


def kernel(features_0, features_1, features_2, x_0, x_1, x_2):
    raise NotImplementedError("write your pallas kernel here")


# speedup vs baseline: 10.7785x; 10.7785x over previous
import functools

import jax
import jax.numpy as jnp
from jax import lax
from jax.experimental import pallas as pl

NC = 80
NA = 3
NO = NC + 5 + 2
BS = 4
GRIDS = [(80, 80), (40, 40), (20, 20)]
FEAT_C = [128, 256, 512]
MAX_DET = 25
IOU_THRES = 0.7
MAX_WH = 7680.0
ANCH = [
    [(1.25, 1.625), (2.0, 3.75), (4.125, 2.875)],
    [(1.875, 3.8125), (3.875, 2.8125), (3.6875, 7.4375)],
    [(3.625, 2.8125), (4.875, 6.1875), (11.65625, 10.1875)],
]
R0, C0 = NA * 80, 80
NSLOT = 32


def _decode_body(level, x_ref, x1_ref, y1_ref, x2_ref, y2_ref, sc_ref):
    ny, nx = GRIDS[level]
    xr = x_ref[0, 0]
    sg = jax.nn.sigmoid(xr)

    obj = sg[..., 4:5]
    cls = sg[..., 5:5 + NC] * obj
    conf4 = jnp.max(cls, axis=-1, keepdims=True)
    li = lax.broadcasted_iota(jnp.int32, (ny, nx, NC), 2).astype(jnp.float32)
    clsi = jnp.min(jnp.where(cls == conf4, li, float(NC)), axis=-1)
    conf = conf4[..., 0]

    col = lax.broadcasted_iota(jnp.int32, (ny, nx), 1).astype(jnp.float32)
    rowy = lax.broadcasted_iota(jnp.int32, (ny, nx), 0).astype(jnp.float32)
    a = pl.program_id(1)
    anc = ANCH[level]
    aw = jnp.where(a == 0, anc[0][0], jnp.where(a == 1, anc[1][0], anc[2][0]))
    ah = jnp.where(a == 0, anc[0][1], jnp.where(a == 1, anc[1][1], anc[2][1]))

    cx = sg[..., 0] * 2.0 + (col - 0.5)
    cy = sg[..., 1] * 2.0 + (rowy - 0.5)
    w = (sg[..., 2] * 2.0) ** 2 * aw
    h = (sg[..., 3] * 2.0) ** 2 * ah
    off = clsi * MAX_WH

    x1_ref[0, 0] = (cx - w / 2.0) + off
    y1_ref[0, 0] = (cy - h / 2.0) + off
    x2_ref[0, 0] = (cx + w / 2.0) + off
    y2_ref[0, 0] = (cy + h / 2.0) + off
    sc_ref[0, 0] = conf


def _decode(level, x):
    ny, nx = GRIDS[level]
    shp = jax.ShapeDtypeStruct((BS, NA, ny, nx), jnp.float32)
    outs = pl.pallas_call(
        functools.partial(_decode_body, level),
        grid=(BS, NA),
        in_specs=[pl.BlockSpec((1, 1, ny, nx, NO),
                               lambda b, a: (b, a, 0, 0, 0))],
        out_specs=[pl.BlockSpec((1, 1, ny, nx),
                                lambda b, a: (b, a, 0, 0))] * 5,
        out_shape=[shp] * 5,
    )(x)
    return [o.reshape(BS, NA * ny, nx) for o in outs]


def _padcat(p0, p1, p2, fill):
    f = jnp.float32(fill)
    r1, c1 = p1.shape[1], p1.shape[2]
    p1 = jnp.concatenate(
        [p1, jnp.full((BS, r1, C0 - c1), f, jnp.float32)], axis=2)
    p1 = jnp.concatenate(
        [p1, jnp.full((BS, R0 - r1, C0), f, jnp.float32)], axis=1)
    r2, c2 = p2.shape[1], p2.shape[2]
    p2 = jnp.concatenate(
        [p2, jnp.full((BS, r2, C0 - c2), f, jnp.float32)], axis=2)
    p2 = jnp.concatenate(
        [p2, jnp.full((BS, R0 - r2, C0), f, jnp.float32)], axis=1)
    return jnp.concatenate([p0, p1, p2], axis=0)


def _nms_body(*refs):
    ins = [r[...] for r in refs[:15]]
    pix_ref, val_ref = refs[15], refs[16]
    FILL = 1e8
    X1 = _padcat(ins[0], ins[5], ins[10], FILL)
    Y1 = _padcat(ins[1], ins[6], ins[11], FILL)
    X2 = _padcat(ins[2], ins[7], ins[12], FILL)
    Y2 = _padcat(ins[3], ins[8], ins[13], FILL)
    S = _padcat(ins[4], ins[9], ins[14], -jnp.inf)
    area = (X2 - X1) * (Y2 - Y1)

    B3 = 3 * BS
    lin = (lax.broadcasted_iota(jnp.int32, (B3, R0, C0), 1) * C0
           + lax.broadcasted_iota(jnp.int32, (B3, R0, C0), 2)).astype(jnp.float32)

    def pixmap(ny, nx):
        pm = (lax.broadcasted_iota(jnp.int32, (BS, NA, ny, nx), 2) * nx
              + lax.broadcasted_iota(jnp.int32, (BS, NA, ny, nx), 3)
              ).astype(jnp.float32)
        return pm.reshape(BS, NA * ny, nx)
    PIX = _padcat(pixmap(80, 80), pixmap(40, 40), pixmap(20, 20), 0.0)

    BIG = jnp.float32(1e9)
    dcol = lax.broadcasted_iota(jnp.int32, (B3, NSLOT), 1)

    def body(d, carry):
        s, selpix, selval = carry
        m = jnp.max(s, axis=(1, 2), keepdims=True)
        valid = (m != -jnp.inf).astype(jnp.float32)
        kk = jnp.min(jnp.where(s == m, lin, BIG), axis=(1, 2), keepdims=True)
        sel = (lin == kk)

        def pick(a):
            return jnp.sum(jnp.where(sel, a, 0.0), axis=(1, 2), keepdims=True)

        x1k, y1k, x2k, y2k = pick(X1), pick(Y1), pick(X2), pick(Y2)
        ak = pick(area)
        iw = jnp.maximum(jnp.minimum(x2k, X2) - jnp.maximum(x1k, X1), 0.0)
        ih = jnp.maximum(jnp.minimum(y2k, Y2) - jnp.maximum(y1k, Y1), 0.0)
        inter = iw * ih
        iou = inter / (ak + area - inter)
        s = jnp.where(iou > IOU_THRES, -jnp.inf, s)
        s = jnp.where(sel, -jnp.inf, s)

        pk = jnp.sum(jnp.where(sel, PIX, 0.0), axis=(1, 2))
        upd = (dcol == d)
        selpix = jnp.where(upd, pk[:, None], selpix)
        selval = jnp.where(upd, valid[:, :, 0], selval)
        return s, selpix, selval

    init = (S, jnp.zeros((B3, NSLOT), jnp.float32),
            jnp.zeros((B3, NSLOT), jnp.float32))
    _, selpix, selval = lax.fori_loop(0, MAX_DET, body, init)
    pix_ref[...] = selpix
    val_ref[...] = selval


def _gather_body(P, feat_ref, pix_ref, val_ref, out_ref):
    pix = pix_ref[...]
    val = val_ref[...]
    io = lax.broadcasted_iota(jnp.int32, (NSLOT, P), 1).astype(jnp.float32)
    for j in range(BS):
        ohj = (io == pix[j][:, None]).astype(jnp.float32) * val[j][:, None]
        out_ref[j] = lax.dot_general(
            ohj, feat_ref[j], (((1,), (1,)), ((), ())),
            preferred_element_type=jnp.float32)


def kernel(features_0, features_1, features_2, x_0, x_1, x_2):
    planes = [_decode(l, x) for l, x in enumerate((x_0, x_1, x_2))]
    ins = [*planes[0], *planes[1], *planes[2]]

    selpix, selval = pl.pallas_call(
        _nms_body,
        out_shape=(jax.ShapeDtypeStruct((3 * BS, NSLOT), jnp.float32),
                   jax.ShapeDtypeStruct((3 * BS, NSLOT), jnp.float32)),
    )(*ins)

    outs = []
    for l, feats in enumerate((features_0, features_1, features_2)):
        ny, nx = GRIDS[l]
        P = ny * nx
        C = FEAT_C[l]
        f = feats.reshape(BS, C, P)
        o = pl.pallas_call(
            functools.partial(_gather_body, P),
            out_shape=jax.ShapeDtypeStruct((BS, NSLOT, C), jnp.float32),
        )(f, selpix[4 * l:4 * l + 4], selval[4 * l:4 * l + 4])
        outs.append(o[:, :MAX_DET, :])
    return tuple(outs)


# speedup vs baseline: 11.0477x; 1.0250x over previous
import functools

import jax
import jax.numpy as jnp
from jax import lax
from jax.experimental import pallas as pl
from jax.experimental.pallas import tpu as pltpu

NC = 80
NA = 3
NO = NC + 5 + 2
BS = 4
GRIDS = [(80, 80), (40, 40), (20, 20)]
FEAT_C = [128, 256, 512]
MAX_DET = 25
IOU_THRES = 0.7
MAX_WH = 7680.0
ANCH = [
    [(1.25, 1.625), (2.0, 3.75), (4.125, 2.875)],
    [(1.875, 3.8125), (3.875, 2.8125), (3.6875, 7.4375)],
    [(3.625, 2.8125), (4.875, 6.1875), (11.65625, 10.1875)],
]
R0, C0 = NA * 80, 80
NSLOT = 32


def _decode_body(level, x_ref, x1_ref, y1_ref, x2_ref, y2_ref, sc_ref):
    ny, nx = GRIDS[level]
    xr = x_ref[0, 0]

    raw = xr[..., 5:5 + NC]
    rmax4 = jnp.max(raw, axis=-1, keepdims=True)
    li = lax.broadcasted_iota(jnp.int32, (ny, nx, NC), 2).astype(jnp.float32)
    clsi = jnp.min(jnp.where(raw == rmax4, li, float(NC)), axis=-1)
    obj = jax.nn.sigmoid(xr[..., 4])
    conf = jax.nn.sigmoid(rmax4[..., 0]) * obj

    col = lax.broadcasted_iota(jnp.int32, (ny, nx), 1).astype(jnp.float32)
    rowy = lax.broadcasted_iota(jnp.int32, (ny, nx), 0).astype(jnp.float32)
    a = pl.program_id(1)
    anc = ANCH[level]
    aw = jnp.where(a == 0, anc[0][0], jnp.where(a == 1, anc[1][0], anc[2][0]))
    ah = jnp.where(a == 0, anc[0][1], jnp.where(a == 1, anc[1][1], anc[2][1]))

    cx = jax.nn.sigmoid(xr[..., 0]) * 2.0 + (col - 0.5)
    cy = jax.nn.sigmoid(xr[..., 1]) * 2.0 + (rowy - 0.5)
    w = (jax.nn.sigmoid(xr[..., 2]) * 2.0) ** 2 * aw
    h = (jax.nn.sigmoid(xr[..., 3]) * 2.0) ** 2 * ah
    off = clsi * MAX_WH

    x1_ref[0, 0] = (cx - w / 2.0) + off
    y1_ref[0, 0] = (cy - h / 2.0) + off
    x2_ref[0, 0] = (cx + w / 2.0) + off
    y2_ref[0, 0] = (cy + h / 2.0) + off
    sc_ref[0, 0] = conf


def _decode(level, x):
    ny, nx = GRIDS[level]
    shp = jax.ShapeDtypeStruct((BS, NA, ny, nx), jnp.float32)
    outs = pl.pallas_call(
        functools.partial(_decode_body, level),
        grid=(BS, NA),
        in_specs=[pl.BlockSpec((1, 1, ny, nx, NO),
                               lambda b, a: (b, a, 0, 0, 0))],
        out_specs=[pl.BlockSpec((1, 1, ny, nx),
                                lambda b, a: (b, a, 0, 0))] * 5,
        out_shape=[shp] * 5,
        compiler_params=pltpu.CompilerParams(
            dimension_semantics=("parallel", "parallel")),
    )(x)
    return [o.reshape(BS, NA * ny, nx) for o in outs]


def _padcat(p0, p1, p2, fill):
    f = jnp.float32(fill)
    r1, c1 = p1.shape[1], p1.shape[2]
    p1 = jnp.concatenate(
        [p1, jnp.full((BS, r1, C0 - c1), f, jnp.float32)], axis=2)
    p1 = jnp.concatenate(
        [p1, jnp.full((BS, R0 - r1, C0), f, jnp.float32)], axis=1)
    r2, c2 = p2.shape[1], p2.shape[2]
    p2 = jnp.concatenate(
        [p2, jnp.full((BS, r2, C0 - c2), f, jnp.float32)], axis=2)
    p2 = jnp.concatenate(
        [p2, jnp.full((BS, R0 - r2, C0), f, jnp.float32)], axis=1)
    return jnp.concatenate([p0, p1, p2], axis=0)


def _nms_body(*refs):
    ins = [r[...] for r in refs[:15]]
    f_refs = refs[15:18]
    out_refs = refs[18:21]
    FILL = 1e8
    X1 = _padcat(ins[0], ins[5], ins[10], FILL)
    Y1 = _padcat(ins[1], ins[6], ins[11], FILL)
    X2 = _padcat(ins[2], ins[7], ins[12], FILL)
    Y2 = _padcat(ins[3], ins[8], ins[13], FILL)
    S = _padcat(ins[4], ins[9], ins[14], -jnp.inf)
    area = (X2 - X1) * (Y2 - Y1)

    B3 = 3 * BS
    lin = (lax.broadcasted_iota(jnp.int32, (B3, R0, C0), 1) * C0
           + lax.broadcasted_iota(jnp.int32, (B3, R0, C0), 2)).astype(jnp.float32)

    def pixmap(ny, nx):
        pm = (lax.broadcasted_iota(jnp.int32, (BS, NA, ny, nx), 2) * nx
              + lax.broadcasted_iota(jnp.int32, (BS, NA, ny, nx), 3)
              ).astype(jnp.float32)
        return pm.reshape(BS, NA * ny, nx)
    PIX = _padcat(pixmap(80, 80), pixmap(40, 40), pixmap(20, 20), 0.0)

    BIG = jnp.float32(1e9)
    dcol = lax.broadcasted_iota(jnp.int32, (B3, NSLOT), 1)

    def body(d, carry):
        s, selpix, selval = carry
        m = jnp.max(s, axis=(1, 2), keepdims=True)
        valid = (m != -jnp.inf).astype(jnp.float32)
        kk = jnp.min(jnp.where(s == m, lin, BIG), axis=(1, 2), keepdims=True)
        sel = (lin == kk)

        def pick(a):
            return jnp.sum(jnp.where(sel, a, 0.0), axis=(1, 2), keepdims=True)

        x1k, y1k, x2k, y2k = pick(X1), pick(Y1), pick(X2), pick(Y2)
        ak = pick(area)
        iw = jnp.maximum(jnp.minimum(x2k, X2) - jnp.maximum(x1k, X1), 0.0)
        ih = jnp.maximum(jnp.minimum(y2k, Y2) - jnp.maximum(y1k, Y1), 0.0)
        inter = iw * ih
        iou = inter / (ak + area - inter)
        s = jnp.where(iou > IOU_THRES, -jnp.inf, s)
        s = jnp.where(sel, -jnp.inf, s)

        pk = jnp.sum(jnp.where(sel, PIX, 0.0), axis=(1, 2))
        upd = (dcol == d)
        selpix = jnp.where(upd, pk[:, None], selpix)
        selval = jnp.where(upd, valid[:, :, 0], selval)
        return s, selpix, selval

    init = (S, jnp.zeros((B3, NSLOT), jnp.float32),
            jnp.zeros((B3, NSLOT), jnp.float32))
    _, selpix, selval = lax.fori_loop(0, MAX_DET, body, init)

    for l in range(3):
        ny, nx = GRIDS[l]
        P = ny * nx
        io = lax.broadcasted_iota(jnp.int32, (NSLOT, P), 1).astype(jnp.float32)
        for j in range(BS):
            s_ = 4 * l + j
            ohj = ((io == selpix[s_][:, None]).astype(jnp.float32)
                   * selval[s_][:, None])
            out_refs[l][j] = lax.dot_general(
                ohj, f_refs[l][j], (((1,), (1,)), ((), ())),
                preferred_element_type=jnp.float32)


def kernel(features_0, features_1, features_2, x_0, x_1, x_2):
    planes = [_decode(l, x) for l, x in enumerate((x_0, x_1, x_2))]
    ins = [*planes[0], *planes[1], *planes[2]]
    feats = [f.reshape(BS, FEAT_C[l], GRIDS[l][0] * GRIDS[l][1])
             for l, f in enumerate((features_0, features_1, features_2))]

    outs = pl.pallas_call(
        _nms_body,
        out_shape=tuple(jax.ShapeDtypeStruct((BS, NSLOT, C), jnp.float32)
                        for C in FEAT_C),
    )(*ins, *feats)
    return tuple(o[:, :MAX_DET, :] for o in outs)


# speedup vs baseline: 15.1091x; 1.3676x over previous
import functools

import jax
import jax.numpy as jnp
from jax import lax
from jax.experimental import pallas as pl
from jax.experimental.pallas import tpu as pltpu

NC = 80
NA = 3
NO = NC + 5 + 2
BS = 4
GRIDS = [(80, 80), (40, 40), (20, 20)]
FEAT_C = [128, 256, 512]
MAX_DET = 25
IOU_THRES = 0.7
MAX_WH = 7680.0
ANCH = [
    [(1.25, 1.625), (2.0, 3.75), (4.125, 2.875)],
    [(1.875, 3.8125), (3.875, 2.8125), (3.6875, 7.4375)],
    [(3.625, 2.8125), (4.875, 6.1875), (11.65625, 10.1875)],
]
R0, C0 = NA * 80, 80
NSLOT = 32


def _decode_body(level, x_ref, x1_ref, y1_ref, x2_ref, y2_ref, sc_ref):
    ny, nx = GRIDS[level]
    xr = x_ref[0, 0]

    raw = xr[5:5 + NC]
    rmax = jnp.max(raw, axis=0)
    li = lax.broadcasted_iota(jnp.int32, (NC, ny, nx), 0).astype(jnp.float32)
    clsi = jnp.min(jnp.where(raw == rmax[None], li, float(NC)), axis=0)
    obj = jax.nn.sigmoid(xr[4])
    conf = jax.nn.sigmoid(rmax) * obj

    col = lax.broadcasted_iota(jnp.int32, (ny, nx), 1).astype(jnp.float32)
    rowy = lax.broadcasted_iota(jnp.int32, (ny, nx), 0).astype(jnp.float32)
    a = pl.program_id(1)
    anc = ANCH[level]
    aw = jnp.where(a == 0, anc[0][0], jnp.where(a == 1, anc[1][0], anc[2][0]))
    ah = jnp.where(a == 0, anc[0][1], jnp.where(a == 1, anc[1][1], anc[2][1]))

    cx = jax.nn.sigmoid(xr[0]) * 2.0 + (col - 0.5)
    cy = jax.nn.sigmoid(xr[1]) * 2.0 + (rowy - 0.5)
    w = (jax.nn.sigmoid(xr[2]) * 2.0) ** 2 * aw
    h = (jax.nn.sigmoid(xr[3]) * 2.0) ** 2 * ah
    off = clsi * MAX_WH

    x1_ref[0, 0] = (cx - w / 2.0) + off
    y1_ref[0, 0] = (cy - h / 2.0) + off
    x2_ref[0, 0] = (cx + w / 2.0) + off
    y2_ref[0, 0] = (cy + h / 2.0) + off
    sc_ref[0, 0] = conf


def _decode(level, x):
    ny, nx = GRIDS[level]
    xt = x.transpose(0, 1, 4, 2, 3)
    shp = jax.ShapeDtypeStruct((BS, NA, ny, nx), jnp.float32)
    outs = pl.pallas_call(
        functools.partial(_decode_body, level),
        grid=(BS, NA),
        in_specs=[pl.BlockSpec((1, 1, NO, ny, nx),
                               lambda b, a: (b, a, 0, 0, 0))],
        out_specs=[pl.BlockSpec((1, 1, ny, nx),
                                lambda b, a: (b, a, 0, 0))] * 5,
        out_shape=[shp] * 5,
        compiler_params=pltpu.CompilerParams(
            dimension_semantics=("parallel", "parallel")),
    )(xt)
    return [o.reshape(BS, NA * ny, nx) for o in outs]


def _pad_rc(p, rows, cols, fill):
    f = jnp.float32(fill)
    r, c = p.shape[1], p.shape[2]
    if cols > c:
        p = jnp.concatenate(
            [p, jnp.full((p.shape[0], r, cols - c), f, jnp.float32)], axis=2)
    if rows > r:
        p = jnp.concatenate(
            [p, jnp.full((p.shape[0], rows - r, cols), f, jnp.float32)],
            axis=1)
    return p


def _nms_group(planes, lin, pixmaps, dcol, nslots):
    X1, Y1, X2, Y2, S = planes
    area = (X2 - X1) * (Y2 - Y1)
    BIG = jnp.float32(1e9)

    def step(d, s, selpix, selval):
        m = jnp.max(s, axis=(1, 2), keepdims=True)
        valid = (m != -jnp.inf).astype(jnp.float32)
        kk = jnp.min(jnp.where(s == m, lin, BIG), axis=(1, 2), keepdims=True)
        sel = (lin == kk)

        def pick(a):
            return jnp.sum(jnp.where(sel, a, 0.0), axis=(1, 2), keepdims=True)

        x1k, y1k, x2k, y2k, ak = pick(X1), pick(Y1), pick(X2), pick(Y2), \
            pick(area)
        iw = jnp.maximum(jnp.minimum(x2k, X2) - jnp.maximum(x1k, X1), 0.0)
        ih = jnp.maximum(jnp.minimum(y2k, Y2) - jnp.maximum(y1k, Y1), 0.0)
        inter = iw * ih
        iou = inter / (ak + area - inter)
        s = jnp.where(iou > IOU_THRES, -jnp.inf, s)
        s = jnp.where(sel, -jnp.inf, s)

        pk = jnp.sum(jnp.where(sel, pixmaps, 0.0), axis=(1, 2))
        upd = (dcol == d)
        selpix = jnp.where(upd, pk[:, None], selpix)
        selval = jnp.where(upd, valid[:, :, 0], selval)
        return s, selpix, selval

    return S, step


def _nms_body(*refs):
    ins = [r[...] for r in refs[:15]]
    f_refs = refs[15:18]
    out_refs = refs[18:21]
    FILL = 1e8

    def pixmap(ny, nx):
        pm = (lax.broadcasted_iota(jnp.int32, (BS, NA, ny, nx), 2) * nx
              + lax.broadcasted_iota(jnp.int32, (BS, NA, ny, nx), 3)
              ).astype(jnp.float32)
        return pm.reshape(BS, NA * ny, nx)

    planesA = ins[0:5]
    pixA = pixmap(80, 80)
    linA = (lax.broadcasted_iota(jnp.int32, (BS, R0, C0), 1) * C0
            + lax.broadcasted_iota(jnp.int32, (BS, R0, C0), 2)
            ).astype(jnp.float32)
    dcolA = lax.broadcasted_iota(jnp.int32, (BS, NSLOT), 1)

    R1, C1 = NA * 40, 40
    planesB = [jnp.concatenate(
        [ins[5 + k], _pad_rc(ins[10 + k], R1, C1, FILL if k < 4 else -jnp.inf)],
        axis=0) for k in range(5)]
    pixB = jnp.concatenate([pixmap(40, 40), _pad_rc(pixmap(20, 20), R1, C1, 0.0)],
                           axis=0)
    linB = (lax.broadcasted_iota(jnp.int32, (2 * BS, R1, C1), 1) * C1
            + lax.broadcasted_iota(jnp.int32, (2 * BS, R1, C1), 2)
            ).astype(jnp.float32)
    dcolB = lax.broadcasted_iota(jnp.int32, (2 * BS, NSLOT), 1)

    SA, stepA = _nms_group(planesA, linA, pixA, dcolA, NSLOT)
    SB, stepB = _nms_group(planesB, linB, pixB, dcolB, NSLOT)

    def body(d, carry):
        sA, pA, vA, sB, pB, vB = carry
        sA, pA, vA = stepA(d, sA, pA, vA)
        sB, pB, vB = stepB(d, sB, pB, vB)
        return sA, pA, vA, sB, pB, vB

    zA = jnp.zeros((BS, NSLOT), jnp.float32)
    zB = jnp.zeros((2 * BS, NSLOT), jnp.float32)
    init = (SA, zA, zA, SB, zB, zB)
    _, pA, vA, sB_, pB, vB = lax.fori_loop(0, MAX_DET, body, init)
    selpix = jnp.concatenate([pA, pB], axis=0)
    selval = jnp.concatenate([vA, vB], axis=0)

    for l in range(3):
        ny, nx = GRIDS[l]
        P = ny * nx
        io = lax.broadcasted_iota(jnp.int32, (NSLOT, P), 1).astype(jnp.float32)
        for j in range(BS):
            s_ = 4 * l + j
            ohj = ((io == selpix[s_][:, None]).astype(jnp.float32)
                   * selval[s_][:, None])
            out_refs[l][j] = lax.dot_general(
                ohj, f_refs[l][j], (((1,), (1,)), ((), ())),
                preferred_element_type=jnp.float32)


def kernel(features_0, features_1, features_2, x_0, x_1, x_2):
    planes = [_decode(l, x) for l, x in enumerate((x_0, x_1, x_2))]
    ins = [*planes[0], *planes[1], *planes[2]]
    feats = [f.reshape(BS, FEAT_C[l], GRIDS[l][0] * GRIDS[l][1])
             for l, f in enumerate((features_0, features_1, features_2))]

    outs = pl.pallas_call(
        _nms_body,
        out_shape=tuple(jax.ShapeDtypeStruct((BS, NSLOT, C), jnp.float32)
                        for C in FEAT_C),
    )(*ins, *feats)
    return tuple(o[:, :MAX_DET, :] for o in outs)


# speedup vs baseline: 18.3061x; 1.2116x over previous
import functools

import jax
import jax.numpy as jnp
from jax import lax
from jax.experimental import pallas as pl
from jax.experimental.pallas import tpu as pltpu

NC = 80
NA = 3
NO = NC + 5 + 2
BS = 4
GRIDS = [(80, 80), (40, 40), (20, 20)]
FEAT_C = [128, 256, 512]
MAX_DET = 25
IOU_THRES = 0.7
MAX_WH = 7680.0
ANCH = [
    [(1.25, 1.625), (2.0, 3.75), (4.125, 2.875)],
    [(1.875, 3.8125), (3.875, 2.8125), (3.6875, 7.4375)],
    [(3.625, 2.8125), (4.875, 6.1875), (11.65625, 10.1875)],
]
R0, C0 = NA * 80, 80
NSLOT = 32


def _decode_body(level, x_ref, x1_ref, y1_ref, x2_ref, y2_ref, sc_ref):
    ny, nx = GRIDS[level]
    x2d = x_ref[0, 0].reshape(ny * nx, NO)
    eye = (lax.broadcasted_iota(jnp.int32, (NO, NO), 0)
           == lax.broadcasted_iota(jnp.int32, (NO, NO), 1)
           ).astype(jnp.float32)
    xt = lax.dot_general(eye, x2d, (((1,), (1,)), ((), ())),
                         preferred_element_type=jnp.float32)
    xr = xt.reshape(NO, ny, nx)

    raw = xr[5:5 + NC]
    rmax = jnp.max(raw, axis=0)
    li = lax.broadcasted_iota(jnp.int32, (NC, ny, nx), 0).astype(jnp.float32)
    clsi = jnp.min(jnp.where(raw == rmax[None], li, float(NC)), axis=0)
    obj = jax.nn.sigmoid(xr[4])
    conf = jax.nn.sigmoid(rmax) * obj

    col = lax.broadcasted_iota(jnp.int32, (ny, nx), 1).astype(jnp.float32)
    rowy = lax.broadcasted_iota(jnp.int32, (ny, nx), 0).astype(jnp.float32)
    a = pl.program_id(1)
    anc = ANCH[level]
    aw = jnp.where(a == 0, anc[0][0], jnp.where(a == 1, anc[1][0], anc[2][0]))
    ah = jnp.where(a == 0, anc[0][1], jnp.where(a == 1, anc[1][1], anc[2][1]))

    cx = jax.nn.sigmoid(xr[0]) * 2.0 + (col - 0.5)
    cy = jax.nn.sigmoid(xr[1]) * 2.0 + (rowy - 0.5)
    w = (jax.nn.sigmoid(xr[2]) * 2.0) ** 2 * aw
    h = (jax.nn.sigmoid(xr[3]) * 2.0) ** 2 * ah
    off = clsi * MAX_WH

    x1_ref[0, 0] = (cx - w / 2.0) + off
    y1_ref[0, 0] = (cy - h / 2.0) + off
    x2_ref[0, 0] = (cx + w / 2.0) + off
    y2_ref[0, 0] = (cy + h / 2.0) + off
    sc_ref[0, 0] = conf


def _decode(level, x):
    ny, nx = GRIDS[level]
    shp = jax.ShapeDtypeStruct((BS, NA, ny, nx), jnp.float32)
    outs = pl.pallas_call(
        functools.partial(_decode_body, level),
        grid=(BS, NA),
        in_specs=[pl.BlockSpec((1, 1, ny, nx, NO),
                               lambda b, a: (b, a, 0, 0, 0))],
        out_specs=[pl.BlockSpec((1, 1, ny, nx),
                                lambda b, a: (b, a, 0, 0))] * 5,
        out_shape=[shp] * 5,
        compiler_params=pltpu.CompilerParams(
            dimension_semantics=("parallel", "parallel")),
    )(x)
    return [o.reshape(BS, NA * ny, nx) for o in outs]


def _pad_rc(p, rows, cols, fill):
    f = jnp.float32(fill)
    r, c = p.shape[1], p.shape[2]
    if cols > c:
        p = jnp.concatenate(
            [p, jnp.full((p.shape[0], r, cols - c), f, jnp.float32)], axis=2)
    if rows > r:
        p = jnp.concatenate(
            [p, jnp.full((p.shape[0], rows - r, cols), f, jnp.float32)],
            axis=1)
    return p


def _nms_group(planes, lin, pixmaps, dcol, nslots):
    X1, Y1, X2, Y2, S = planes
    area = (X2 - X1) * (Y2 - Y1)
    BIG = jnp.float32(1e9)

    def step(d, s, selpix, selval):
        m = jnp.max(s, axis=(1, 2), keepdims=True)
        valid = (m != -jnp.inf).astype(jnp.float32)
        kk = jnp.min(jnp.where(s == m, lin, BIG), axis=(1, 2), keepdims=True)
        sel = (lin == kk)

        def pick(a):
            return jnp.sum(jnp.where(sel, a, 0.0), axis=(1, 2), keepdims=True)

        x1k, y1k, x2k, y2k, ak = pick(X1), pick(Y1), pick(X2), pick(Y2), \
            pick(area)
        iw = jnp.maximum(jnp.minimum(x2k, X2) - jnp.maximum(x1k, X1), 0.0)
        ih = jnp.maximum(jnp.minimum(y2k, Y2) - jnp.maximum(y1k, Y1), 0.0)
        inter = iw * ih
        iou = inter / (ak + area - inter)
        s = jnp.where(iou > IOU_THRES, -jnp.inf, s)
        s = jnp.where(sel, -jnp.inf, s)

        pk = jnp.sum(jnp.where(sel, pixmaps, 0.0), axis=(1, 2))
        upd = (dcol == d)
        selpix = jnp.where(upd, pk[:, None], selpix)
        selval = jnp.where(upd, valid[:, :, 0], selval)
        return s, selpix, selval

    return S, step


def _nms_body(*refs):
    ins = [r[...] for r in refs[:15]]
    f_refs = refs[15:18]
    out_refs = refs[18:21]
    FILL = 1e8

    def pixmap(ny, nx):
        pm = (lax.broadcasted_iota(jnp.int32, (BS, NA, ny, nx), 2) * nx
              + lax.broadcasted_iota(jnp.int32, (BS, NA, ny, nx), 3)
              ).astype(jnp.float32)
        return pm.reshape(BS, NA * ny, nx)

    planesA = ins[0:5]
    pixA = pixmap(80, 80)
    linA = (lax.broadcasted_iota(jnp.int32, (BS, R0, C0), 1) * C0
            + lax.broadcasted_iota(jnp.int32, (BS, R0, C0), 2)
            ).astype(jnp.float32)
    dcolA = lax.broadcasted_iota(jnp.int32, (BS, NSLOT), 1)

    R1, C1 = NA * 40, 40
    planesB = [jnp.concatenate(
        [ins[5 + k], _pad_rc(ins[10 + k], R1, C1, FILL if k < 4 else -jnp.inf)],
        axis=0) for k in range(5)]
    pixB = jnp.concatenate([pixmap(40, 40), _pad_rc(pixmap(20, 20), R1, C1, 0.0)],
                           axis=0)
    linB = (lax.broadcasted_iota(jnp.int32, (2 * BS, R1, C1), 1) * C1
            + lax.broadcasted_iota(jnp.int32, (2 * BS, R1, C1), 2)
            ).astype(jnp.float32)
    dcolB = lax.broadcasted_iota(jnp.int32, (2 * BS, NSLOT), 1)

    SA, stepA = _nms_group(planesA, linA, pixA, dcolA, NSLOT)
    SB, stepB = _nms_group(planesB, linB, pixB, dcolB, NSLOT)

    def body(d, carry):
        sA, pA, vA, sB, pB, vB = carry
        sA, pA, vA = stepA(d, sA, pA, vA)
        sB, pB, vB = stepB(d, sB, pB, vB)
        return sA, pA, vA, sB, pB, vB

    zA = jnp.zeros((BS, NSLOT), jnp.float32)
    zB = jnp.zeros((2 * BS, NSLOT), jnp.float32)
    init = (SA, zA, zA, SB, zB, zB)
    _, pA, vA, sB_, pB, vB = lax.fori_loop(0, MAX_DET, body, init)
    selpix = jnp.concatenate([pA, pB], axis=0)
    selval = jnp.concatenate([vA, vB], axis=0)

    for l in range(3):
        ny, nx = GRIDS[l]
        P = ny * nx
        io = lax.broadcasted_iota(jnp.int32, (NSLOT, P), 1).astype(jnp.float32)
        for j in range(BS):
            s_ = 4 * l + j
            ohj = ((io == selpix[s_][:, None]).astype(jnp.float32)
                   * selval[s_][:, None])
            out_refs[l][j] = lax.dot_general(
                ohj, f_refs[l][j], (((1,), (1,)), ((), ())),
                preferred_element_type=jnp.float32)


def kernel(features_0, features_1, features_2, x_0, x_1, x_2):
    planes = [_decode(l, x) for l, x in enumerate((x_0, x_1, x_2))]
    ins = [*planes[0], *planes[1], *planes[2]]
    feats = [f.reshape(BS, FEAT_C[l], GRIDS[l][0] * GRIDS[l][1])
             for l, f in enumerate((features_0, features_1, features_2))]

    outs = pl.pallas_call(
        _nms_body,
        out_shape=tuple(jax.ShapeDtypeStruct((BS, NSLOT, C), jnp.float32)
                        for C in FEAT_C),
    )(*ins, *feats)
    return tuple(o[:, :MAX_DET, :] for o in outs)
